# initial kernel scaffold (unmeasured)
import jax
import jax.numpy as jnp
from jax import lax
from jax.experimental import pallas as pl
from jax.experimental.pallas import tpu as pltpu

N_DEV = 8


def kernel(x, w_mat):
    k_glob, k_per = x.shape
    _, n = w_mat.shape
    blk = k_glob // N_DEV

    def body(x_ref, w_ref, out_ref, comm_ref, send_sems, recv_sems):
        me = lax.axis_index("i")

        barrier_sem = pltpu.get_barrier_semaphore()
        for j in range(N_DEV):
            @pl.when(me != j)
            def _():
                pl.semaphore_signal(
                    barrier_sem, inc=1,
                    device_id=(j,), device_id_type=pl.DeviceIdType.MESH,
                )
        pl.semaphore_wait(barrier_sem, N_DEV - 1)

        for j in range(N_DEV):
            @pl.when(me != j)
            def _():
                rdma = pltpu.make_async_remote_copy(
                    src_ref=x_ref.at[pl.ds(j * blk, blk), :],
                    dst_ref=comm_ref.at[me],
                    send_sem=send_sems.at[j],
                    recv_sem=recv_sems.at[me],
                    device_id=(j,),
                    device_id_type=pl.DeviceIdType.MESH,
                )
                rdma.start()

        out_ref[...] = jnp.dot(
            x_ref[pl.ds(me * blk, blk), :],
            w_ref[pl.ds(me * blk, blk), :],
            preferred_element_type=jnp.float32,
        )

        for src in range(N_DEV):
            @pl.when(me != src)
            def _():
                recv = pltpu.make_async_remote_copy(
                    src_ref=x_ref.at[pl.ds(0, blk), :],
                    dst_ref=comm_ref.at[src],
                    send_sem=send_sems.at[src],
                    recv_sem=recv_sems.at[src],
                    device_id=(src,),
                    device_id_type=pl.DeviceIdType.MESH,
                )
                recv.wait_recv()
                out_ref[...] += jnp.dot(
                    comm_ref[src],
                    w_ref[pl.ds(src * blk, blk), :],
                    preferred_element_type=jnp.float32,
                )

        for j in range(N_DEV):
            @pl.when(me != j)
            def _():
                snd = pltpu.make_async_remote_copy(
                    src_ref=x_ref.at[pl.ds(j * blk, blk), :],
                    dst_ref=comm_ref.at[0],
                    send_sem=send_sems.at[j],
                    recv_sem=recv_sems.at[0],
                    device_id=(j,),
                    device_id_type=pl.DeviceIdType.MESH,
                )
                snd.wait_send()

        y = out_ref[...]
        out_ref[...] = y * jax.nn.sigmoid(y)

    return pl.pallas_call(
        body,
        out_shape=jax.ShapeDtypeStruct((blk, n), jnp.float32),
        in_specs=[
            pl.BlockSpec(memory_space=pltpu.VMEM),
            pl.BlockSpec(memory_space=pltpu.VMEM),
        ],
        out_specs=pl.BlockSpec(memory_space=pltpu.VMEM),
        scratch_shapes=[
            pltpu.VMEM((N_DEV, blk, k_per), jnp.float32),
            pltpu.SemaphoreType.DMA((N_DEV,)),
            pltpu.SemaphoreType.DMA((N_DEV,)),
        ],
        compiler_params=pltpu.CompilerParams(collective_id=0),
    )(x, w_mat)


# baseline (device time: 89405 ns/iter reference)
import jax
import jax.numpy as jnp
from jax import lax
from jax.experimental import pallas as pl
from jax.experimental.pallas import tpu as pltpu

N_DEV = 8


def kernel(x, w_mat):
    k_glob, k_per = x.shape
    _, n = w_mat.shape
    blk = k_glob // N_DEV

    def body(x_ref, w_ref, out_ref, comm_ref, send_sems, recv_sems):
        me = lax.axis_index("i")

        barrier_sem = pltpu.get_barrier_semaphore()
        for j in range(N_DEV):
            @pl.when(me != j)
            def _():
                pl.semaphore_signal(
                    barrier_sem, inc=1,
                    device_id=(j,), device_id_type=pl.DeviceIdType.MESH,
                )
        pl.semaphore_wait(barrier_sem, N_DEV - 1)

        for j in range(N_DEV):
            @pl.when(me != j)
            def _():
                rdma = pltpu.make_async_remote_copy(
                    src_ref=x_ref.at[pl.ds(j * blk, blk), :],
                    dst_ref=comm_ref.at[me],
                    send_sem=send_sems.at[j],
                    recv_sem=recv_sems.at[me],
                    device_id=(j,),
                    device_id_type=pl.DeviceIdType.MESH,
                )
                rdma.start()

        out_ref[...] = jnp.dot(
            x_ref[pl.ds(me * blk, blk), :],
            w_ref[pl.ds(me * blk, blk), :],
            preferred_element_type=jnp.float32,
        )

        for src in range(N_DEV):
            @pl.when(me != src)
            def _():
                recv = pltpu.make_async_remote_copy(
                    src_ref=x_ref.at[pl.ds(0, blk), :],
                    dst_ref=comm_ref.at[src],
                    send_sem=send_sems.at[src],
                    recv_sem=recv_sems.at[src],
                    device_id=(src,),
                    device_id_type=pl.DeviceIdType.MESH,
                )
                recv.wait_recv()
                out_ref[...] += jnp.dot(
                    comm_ref[src],
                    w_ref[pl.ds(src * blk, blk), :],
                    preferred_element_type=jnp.float32,
                )

        for j in range(N_DEV):
            @pl.when(me != j)
            def _():
                snd = pltpu.make_async_remote_copy(
                    src_ref=x_ref.at[pl.ds(j * blk, blk), :],
                    dst_ref=comm_ref.at[0],
                    send_sem=send_sems.at[j],
                    recv_sem=recv_sems.at[0],
                    device_id=(j,),
                    device_id_type=pl.DeviceIdType.MESH,
                )
                snd.wait_send()

        y = out_ref[...]
        out_ref[...] = y * jax.nn.sigmoid(y)

    return pl.pallas_call(
        body,
        out_shape=jax.ShapeDtypeStruct((blk, n), jnp.float32),
        in_specs=[
            pl.BlockSpec(memory_space=pltpu.VMEM),
            pl.BlockSpec(memory_space=pltpu.VMEM),
        ],
        out_specs=pl.BlockSpec(memory_space=pltpu.VMEM),
        scratch_shapes=[
            pltpu.VMEM((N_DEV, blk, k_per), jnp.float32),
            pltpu.SemaphoreType.DMA((N_DEV,)),
            pltpu.SemaphoreType.DMA((N_DEV,)),
        ],
        compiler_params=pltpu.CompilerParams(
            collective_id=0,
            vmem_limit_bytes=100 * 1024 * 1024,
        ),
    )(x, w_mat)
